# baseline (speedup 1.0000x reference)
import functools

import jax
import jax.numpy as jnp
from jax import lax
from jax.experimental import pallas as pl
from jax.experimental.pallas import tpu as pltpu
from jax.experimental.pallas import tpu_sc as plsc

_D = 64
_NW = 32
_CHUNK = 512


def _make_gather(b_tot: int):
    b_per_w = b_tot // _NW
    n_chunks = b_per_w // _CHUNK
    mesh = plsc.VectorSubcoreMesh(core_axis_name="c", subcore_axis_name="s")

    @functools.partial(
        pl.kernel,
        mesh=mesh,
        out_type=jax.ShapeDtypeStruct((b_tot, _D), jnp.float32),
        scratch_types=[
            pltpu.VMEM((_CHUNK,), jnp.int32),
            pltpu.VMEM((_CHUNK, _D), jnp.float32),
            pltpu.SemaphoreType.DMA,
        ],
        compiler_params=pltpu.CompilerParams(use_tc_tiling_on_sc=False),
    )
    def gather(idx_hbm, table_hbm, out_hbm, idx_v, rows_v, gsem):
        wid = lax.axis_index("s") * 2 + lax.axis_index("c")
        base = wid * b_per_w

        def body(i, carry):
            off = base + i * _CHUNK
            pltpu.sync_copy(idx_hbm.at[pl.ds(off, _CHUNK)], idx_v)
            pltpu.async_copy(table_hbm.at[idx_v], rows_v, gsem).wait()
            pltpu.sync_copy(rows_v, out_hbm.at[pl.ds(off, _CHUNK)])
            return carry

        lax.fori_loop(0, n_chunks, body, 0)

    return gather


def kernel(token_ids, table):
    b, s = token_ids.shape
    flat = token_ids.reshape(b * s).astype(jnp.int32)
    out = _make_gather(b * s)(flat, table)
    return out.reshape(b, s, _D)


# speedup vs baseline: 1.0456x; 1.0456x over previous
import functools

import jax
import jax.numpy as jnp
from jax import lax
from jax.experimental import pallas as pl
from jax.experimental.pallas import tpu as pltpu
from jax.experimental.pallas import tpu_sc as plsc

_D = 64
_NW = 32
_CHUNK = 400
_NBUF = 4


def _make_gather(b_tot: int):
    b_per_w = b_tot // _NW
    n_chunks = b_per_w // _CHUNK
    n_groups = n_chunks // _NBUF
    mesh = plsc.VectorSubcoreMesh(core_axis_name="c", subcore_axis_name="s")

    @functools.partial(
        pl.kernel,
        mesh=mesh,
        out_type=jax.ShapeDtypeStruct((b_tot, _D), jnp.float32),
        scratch_types=[
            [pltpu.VMEM((_CHUNK,), jnp.int32)] * _NBUF,
            [pltpu.VMEM((_CHUNK, _D), jnp.float32)] * _NBUF,
            [pltpu.SemaphoreType.DMA] * _NBUF,
            [pltpu.SemaphoreType.DMA] * _NBUF,
        ],
        compiler_params=pltpu.CompilerParams(use_tc_tiling_on_sc=False),
    )
    def gather(idx_hbm, table_hbm, out_hbm, idx_v, rows_v, gsems, ssems):
        wid = lax.axis_index("s") * 2 + lax.axis_index("c")
        base = wid * b_per_w

        def stage_and_fire(i, b):
            pltpu.sync_copy(
                idx_hbm.at[pl.ds(base + i * _CHUNK, _CHUNK)], idx_v[b]
            )
            pltpu.async_copy(table_hbm.at[idx_v[b]], rows_v[b], gsems[b])

        for b in range(_NBUF):
            stage_and_fire(b, b)

        def body(g, carry):
            for b in range(_NBUF):
                i = g * _NBUF + b
                pltpu.make_async_copy(
                    table_hbm.at[idx_v[b]], rows_v[b], gsems[b]
                ).wait()
                pltpu.async_copy(
                    rows_v[b],
                    out_hbm.at[pl.ds(base + i * _CHUNK, _CHUNK)],
                    ssems[b],
                )

                @pl.when(g < n_groups - 1)
                def _():
                    pltpu.sync_copy(
                        idx_hbm.at[pl.ds(base + (i + _NBUF) * _CHUNK, _CHUNK)],
                        idx_v[b],
                    )
                    pltpu.make_async_copy(
                        rows_v[b],
                        out_hbm.at[pl.ds(base + i * _CHUNK, _CHUNK)],
                        ssems[b],
                    ).wait()
                    pltpu.async_copy(
                        table_hbm.at[idx_v[b]], rows_v[b], gsems[b]
                    )

            return carry

        lax.fori_loop(0, n_groups, body, 0)

        for b in range(_NBUF):
            pltpu.make_async_copy(
                rows_v[b], out_hbm.at[pl.ds(base, _CHUNK)], ssems[b]
            ).wait()

    return gather


def kernel(token_ids, table):
    b, s = token_ids.shape
    flat = token_ids.reshape(b * s).astype(jnp.int32)
    out = _make_gather(b * s)(flat, table)
    return out.reshape(b, s, _D)
